# Initial kernel scaffold; baseline (speedup 1.0000x reference)
import functools

import jax
import jax.numpy as jnp
from jax import lax
from jax.experimental import pallas as pl
from jax.experimental.pallas import tpu as pltpu
from jax.experimental.pallas import tpu_sc as plsc

SIGMA = 1.0
EPSILON = 1.0
CUTOFF = 2.5

NUM_CORES = 2
NUM_SUBCORES = 16
NUM_TILES = NUM_CORES * NUM_SUBCORES
LANES = 16
C_EDGES = 2048
K_SUB = C_EDGES // 128


def _round_up(x, m):
    return (x + m - 1) // m * m


@functools.partial(jax.jit, static_argnames=("n_pad", "n_chunks"))
def _lj_call(pos_pad, zeros_pad, mi2d, mj2d, n_pad, n_chunks):
    rows_stage = n_pad // NUM_SUBCORES

    def body(pos_hbm, zf_hbm, mi_hbm, mj_hbm, fpart, epart,
             pos_sp, f_sp, ibuf, jbuf, pbi, pbj, fbi, fbj, ev, sem):
        cid = lax.axis_index("c")
        sid = lax.axis_index("s")
        wid = cid * NUM_SUBCORES + sid
        r0 = sid * rows_stage
        pltpu.sync_copy(pos_hbm.at[pl.ds(r0, rows_stage)],
                        pos_sp.at[pl.ds(r0, rows_stage)])
        pltpu.sync_copy(zf_hbm.at[pl.ds(r0, rows_stage)],
                        f_sp.at[pl.ds(r0, rows_stage)])
        ev[...] = jnp.zeros((LANES,), jnp.float32)
        plsc.subcore_barrier()

        lane = lax.iota(jnp.int32, (LANES,))

        def chunk_body(c, _):
            row0 = (wid * n_chunks + c) * K_SUB
            pltpu.sync_copy(mi_hbm.at[pl.ds(row0, K_SUB)], ibuf)
            pltpu.sync_copy(mj_hbm.at[pl.ds(row0, K_SUB)], jbuf)
            descs = []
            for s in range(K_SUB):
                descs.append(pltpu.async_copy(pos_sp.at[ibuf.at[s]],
                                              pbi.at[s], sem))
                descs.append(pltpu.async_copy(pos_sp.at[jbuf.at[s]],
                                              pbj.at[s], sem))
            for d in descs:
                d.wait()

            def grp(g, _):
                s = g // 8
                r = (g % 8) * LANES
                sv = jnp.full((LANES,), s, jnp.int32)
                rv = r + lane
                c0 = jnp.zeros((LANES,), jnp.int32)
                c1 = jnp.full((LANES,), 1, jnp.int32)
                c2 = jnp.full((LANES,), 2, jnp.int32)
                xi = plsc.load_gather(pbi, [sv, rv, c0])
                yi = plsc.load_gather(pbi, [sv, rv, c1])
                zi = plsc.load_gather(pbi, [sv, rv, c2])
                xj = plsc.load_gather(pbj, [sv, rv, c0])
                yj = plsc.load_gather(pbj, [sv, rv, c1])
                zj = plsc.load_gather(pbj, [sv, rv, c2])
                dx = xj - xi
                dy = yj - yi
                dz = zj - zi
                d2 = dx * dx + dy * dy + dz * dz
                valid = (d2 > 0.0) & (d2 < CUTOFF * CUTOFF)
                d2s = jnp.where(valid, d2, 1.0)
                inv = 1.0 / d2s
                inv3 = inv * inv * inv
                inv6 = inv3 * inv3
                e = jnp.where(valid, 4.0 * EPSILON * (inv6 - inv3), 0.0)
                ev[...] = ev[...] + e
                fs = jnp.where(valid,
                               (24.0 * EPSILON * inv) * (2.0 * inv6 - inv3),
                               0.0)
                fx = fs * dx
                fy = fs * dy
                fz = fs * dz
                plsc.store_scatter(fbj, [sv, rv, c0], fx)
                plsc.store_scatter(fbj, [sv, rv, c1], fy)
                plsc.store_scatter(fbj, [sv, rv, c2], fz)
                plsc.store_scatter(fbi, [sv, rv, c0], -fx)
                plsc.store_scatter(fbi, [sv, rv, c1], -fy)
                plsc.store_scatter(fbi, [sv, rv, c2], -fz)
                return 0

            lax.fori_loop(0, C_EDGES // LANES, grp, 0)
            for s in range(K_SUB):
                pltpu.sync_copy(fbi.at[s], f_sp.at[ibuf.at[s]], add=True)
                pltpu.sync_copy(fbj.at[s], f_sp.at[jbuf.at[s]], add=True)
            return 0

        lax.fori_loop(0, n_chunks, chunk_body, 0)
        plsc.subcore_barrier()
        pltpu.sync_copy(f_sp.at[pl.ds(r0, rows_stage)],
                        fpart.at[cid, pl.ds(r0, rows_stage)])
        pltpu.sync_copy(ev, epart.at[cid, sid])

    mesh = plsc.VectorSubcoreMesh(core_axis_name="c", subcore_axis_name="s")
    fpart, epart = pl.kernel(
        body,
        out_type=[
            jax.ShapeDtypeStruct((NUM_CORES, n_pad, 3), jnp.float32),
            jax.ShapeDtypeStruct((NUM_CORES, NUM_SUBCORES, LANES),
                                 jnp.float32),
        ],
        mesh=mesh,
        scratch_types=[
            pltpu.VMEM_SHARED((n_pad, 3), jnp.float32),
            pltpu.VMEM_SHARED((n_pad, 3), jnp.float32),
            pltpu.VMEM((K_SUB, 128), jnp.int32),
            pltpu.VMEM((K_SUB, 128), jnp.int32),
            pltpu.VMEM((K_SUB, 128, 3), jnp.float32),
            pltpu.VMEM((K_SUB, 128, 3), jnp.float32),
            pltpu.VMEM((K_SUB, 128, 3), jnp.float32),
            pltpu.VMEM((K_SUB, 128, 3), jnp.float32),
            pltpu.VMEM((LANES,), jnp.float32),
            pltpu.SemaphoreType.DMA,
        ],
    )(pos_pad, zeros_pad, mi2d, mj2d)
    return fpart, epart


def kernel(positions, mapping):
    n = positions.shape[0]
    n_edges = mapping.shape[1]
    n_pad = _round_up(n, 128)
    per_tile = C_EDGES * ((n_edges + NUM_TILES * C_EDGES - 1)
                          // (NUM_TILES * C_EDGES))
    n_chunks = per_tile // C_EDGES
    e_pad = NUM_TILES * per_tile

    pos_pad = jnp.zeros((n_pad, 3), jnp.float32).at[:n].set(positions)
    zeros_pad = jnp.zeros((n_pad, 3), jnp.float32)
    mi = jnp.zeros((e_pad,), jnp.int32).at[:n_edges].set(mapping[0])
    mj = jnp.zeros((e_pad,), jnp.int32).at[:n_edges].set(mapping[1])
    mi2d = mi.reshape(e_pad // 128, 128)
    mj2d = mj.reshape(e_pad // 128, 128)

    fpart, epart = _lj_call(pos_pad, zeros_pad, mi2d, mj2d, n_pad, n_chunks)
    energy = 0.5 * jnp.sum(epart)
    forces = (fpart[0] + fpart[1])[:n]
    return (energy, forces)


# speedup vs baseline: 45.0929x; 45.0929x over previous
import functools

import jax
import jax.numpy as jnp
from jax import lax
from jax.experimental import pallas as pl
from jax.experimental.pallas import tpu as pltpu
from jax.experimental.pallas import tpu_sc as plsc

SIGMA = 1.0
EPSILON = 1.0
CUTOFF = 2.5

NUM_CORES = 2
NUM_SUBCORES = 16
NUM_TILES = NUM_CORES * NUM_SUBCORES
LANES = 16
C_EDGES = 2048
K_SUB = C_EDGES // 128
GRPS = C_EDGES // LANES


def _round_up(x, m):
    return (x + m - 1) // m * m


@functools.partial(jax.jit, static_argnames=("n_pad", "n_chunks"))
def _lj_call(px, py, pz, zf, mi2d, mj2d, n_pad, n_chunks):
    rows_stage = n_pad // NUM_SUBCORES

    def body(*refs):
        (px_hbm, py_hbm, pz_hbm, zf_hbm, mi_hbm, mj_hbm, fpart, epart,
         px_sp, py_sp, pz_sp, fx_sp, fy_sp, fz_sp,
         ib0, jb0, ib1, jb1,
         xi0, yi0, zi0, xj0, yj0, zj0,
         xi1, yi1, zi1, xj1, yj1, zj1,
         fxi0, fyi0, fzi0, fxj0, fyj0, fzj0,
         fxi1, fyi1, fzi1, fxj1, fyj1, fzj1,
         ev, bounce, gsem, ssem) = refs
        ibs = (ib0, ib1)
        jbs = (jb0, jb1)
        gbs = ((xi0, yi0, zi0, xj0, yj0, zj0),
               (xi1, yi1, zi1, xj1, yj1, zj1))
        fbs = ((fxi0, fyi0, fzi0, fxj0, fyj0, fzj0),
               (fxi1, fyi1, fzi1, fxj1, fyj1, fzj1))

        cid = lax.axis_index("c")
        sid = lax.axis_index("s")
        wid = cid * NUM_SUBCORES + sid
        r0 = sid * rows_stage
        sl = pl.ds(r0, rows_stage)
        for src_hbm, dst_sp in ((px_hbm, px_sp), (py_hbm, py_sp),
                                (pz_hbm, pz_sp), (zf_hbm, fx_sp),
                                (zf_hbm, fy_sp), (zf_hbm, fz_sp)):
            pltpu.sync_copy(src_hbm.at[sl], bounce)
            pltpu.sync_copy(bounce, dst_sp.at[sl])
        ev[...] = jnp.zeros((LANES,), jnp.float32)
        plsc.subcore_barrier()

        def fetch_fire(c, b):
            ebase = (wid * n_chunks + c) * C_EDGES
            pltpu.sync_copy(mi_hbm.at[pl.ds(ebase, C_EDGES)], ibs[b])
            pltpu.sync_copy(mj_hbm.at[pl.ds(ebase, C_EDGES)], jbs[b])
            xbi, ybi, zbi, xbj, ybj, zbj = gbs[b]
            pltpu.async_copy(px_sp.at[ibs[b]], xbi, gsem)
            pltpu.async_copy(py_sp.at[ibs[b]], ybi, gsem)
            pltpu.async_copy(pz_sp.at[ibs[b]], zbi, gsem)
            pltpu.async_copy(px_sp.at[jbs[b]], xbj, gsem)
            pltpu.async_copy(py_sp.at[jbs[b]], ybj, gsem)
            pltpu.async_copy(pz_sp.at[jbs[b]], zbj, gsem)

        def drain_gathers(b):
            xbi, ybi, zbi, xbj, ybj, zbj = gbs[b]
            for dst in (xbi, ybi, zbi, xbj, ybj, zbj):
                pltpu.make_async_copy(px_sp.at[ibs[b]], dst, gsem).wait()

        def fire_scatters(b):
            fxi, fyi, fzi, fxj, fyj, fzj = fbs[b]
            pltpu.async_copy(fxi, fx_sp.at[ibs[b]], ssem, add=True)
            pltpu.async_copy(fyi, fy_sp.at[ibs[b]], ssem, add=True)
            pltpu.async_copy(fzi, fz_sp.at[ibs[b]], ssem, add=True)
            pltpu.async_copy(fxj, fx_sp.at[jbs[b]], ssem, add=True)
            pltpu.async_copy(fyj, fy_sp.at[jbs[b]], ssem, add=True)
            pltpu.async_copy(fzj, fz_sp.at[jbs[b]], ssem, add=True)

        def drain_scatters(b):
            fxi, fyi, fzi, fxj, fyj, fzj = fbs[b]
            pltpu.make_async_copy(fxi, fx_sp.at[ibs[b]], ssem).wait()
            pltpu.make_async_copy(fyi, fy_sp.at[ibs[b]], ssem).wait()
            pltpu.make_async_copy(fzi, fz_sp.at[ibs[b]], ssem).wait()
            pltpu.make_async_copy(fxj, fx_sp.at[jbs[b]], ssem).wait()
            pltpu.make_async_copy(fyj, fy_sp.at[jbs[b]], ssem).wait()
            pltpu.make_async_copy(fzj, fz_sp.at[jbs[b]], ssem).wait()

        def compute(b):
            xbi, ybi, zbi, xbj, ybj, zbj = gbs[b]
            fxi, fyi, fzi, fxj, fyj, fzj = fbs[b]

            def grp(g, _):
                v = pl.ds(g * LANES, LANES)
                dx = xbj[v] - xbi[v]
                dy = ybj[v] - ybi[v]
                dz = zbj[v] - zbi[v]
                d2 = dx * dx + dy * dy + dz * dz
                valid = (d2 > 0.0) & (d2 < CUTOFF * CUTOFF)
                d2s = jnp.where(valid, d2, 1.0)
                inv = 1.0 / d2s
                inv3 = inv * inv * inv
                inv6 = inv3 * inv3
                e = jnp.where(valid, 4.0 * EPSILON * (inv6 - inv3), 0.0)
                ev[...] = ev[...] + e
                fs = jnp.where(valid,
                               (24.0 * EPSILON * inv) * (2.0 * inv6 - inv3),
                               0.0)
                fx = fs * dx
                fy = fs * dy
                fz = fs * dz
                fxj[v] = fx
                fyj[v] = fy
                fzj[v] = fz
                fxi[v] = -fx
                fyi[v] = -fy
                fzi[v] = -fz
                return 0

            lax.fori_loop(0, GRPS, grp, 0)

        fetch_fire(0, 0)

        @pl.loop(0, n_chunks, step=2)
        def _pair(c):
            for b in (0, 1):
                cc = c + b
                o = 1 - b

                @pl.when(cc + 1 < n_chunks)
                def _():
                    @pl.when(cc >= 1)
                    def _():
                        drain_scatters(o)
                    fetch_fire(cc + 1, o)

                drain_gathers(b)
                compute(b)
                fire_scatters(b)

        drain_scatters(0)
        drain_scatters(1)
        plsc.subcore_barrier()
        base = cid * 3 * n_pad + r0
        for k, src_sp in enumerate((fx_sp, fy_sp, fz_sp)):
            pltpu.sync_copy(src_sp.at[sl], bounce)
            pltpu.sync_copy(bounce, fpart.at[pl.ds(base + k * n_pad,
                                                   rows_stage)])
        pltpu.sync_copy(ev, epart.at[pl.ds(wid * LANES, LANES)])

    mesh = plsc.VectorSubcoreMesh(core_axis_name="c", subcore_axis_name="s")
    fpart, epart = pl.kernel(
        body,
        out_type=[
            jax.ShapeDtypeStruct((NUM_CORES * 3 * n_pad,), jnp.float32),
            jax.ShapeDtypeStruct((NUM_TILES * LANES,), jnp.float32),
        ],
        mesh=mesh,
        scratch_types=(
            [pltpu.VMEM_SHARED((n_pad,), jnp.float32)] * 6
            + [pltpu.VMEM((C_EDGES,), jnp.int32)] * 4
            + [pltpu.VMEM((C_EDGES,), jnp.float32)] * 24
            + [pltpu.VMEM((LANES,), jnp.float32),
               pltpu.VMEM((n_pad // NUM_SUBCORES,), jnp.float32),
               pltpu.SemaphoreType.DMA,
               pltpu.SemaphoreType.DMA]
        ),
    )(px, py, pz, zf, mi2d, mj2d)
    return fpart, epart


def kernel(positions, mapping):
    n = positions.shape[0]
    n_edges = mapping.shape[1]
    n_pad = _round_up(n, 128)
    pair = 2 * NUM_TILES * C_EDGES
    e_pad = pair * ((n_edges + pair - 1) // pair)
    n_chunks = e_pad // (NUM_TILES * C_EDGES)

    pos_pad = jnp.zeros((3, n_pad), jnp.float32).at[:, :n].set(positions.T)
    zf = jnp.zeros((n_pad,), jnp.float32)
    mi = jnp.zeros((e_pad,), jnp.int32).at[:n_edges].set(mapping[0])
    mj = jnp.zeros((e_pad,), jnp.int32).at[:n_edges].set(mapping[1])

    fpart, epart = _lj_call(pos_pad[0], pos_pad[1], pos_pad[2], zf,
                            mi, mj, n_pad, n_chunks)
    energy = 0.5 * jnp.sum(epart)
    fp = fpart.reshape(NUM_CORES, 3, n_pad)
    forces = (fp[0] + fp[1]).T[:n]
    return (energy, forces)
